# Initial kernel scaffold; baseline (speedup 1.0000x reference)
import jax
import jax.numpy as jnp
from jax.experimental import pallas as pl


def kernel(slot_scores, write_score):
    raise NotImplementedError("write your pallas kernel here")


# speedup vs baseline: 2.8835x; 2.8835x over previous
import jax
import jax.numpy as jnp
from jax.experimental import pallas as pl

_K = 64
_BLK = 8


def _select_kernel(x_ref, w_ref, op_ref, wm_ref):
    x = x_ref[...]
    w = w_ref[...]
    blk, n = x.shape

    s = jax.lax.bitcast_convert_type(x, jnp.int32)
    key = jnp.where(s < 0, s ^ jnp.int32(0x7FFFFFFF), s)
    sw = jax.lax.bitcast_convert_type(w, jnp.int32)
    keyw = jnp.where(sw < 0, sw ^ jnp.int32(0x7FFFFFFF), sw)

    row_max = jnp.maximum(jnp.max(key, axis=1, keepdims=True), keyw)
    row_min = jnp.minimum(jnp.min(key, axis=1, keepdims=True), keyw)
    lo0 = row_min
    hi0 = row_max + 1

    def vcond(c):
        lo, hi = c
        return jnp.any(hi > lo + 1)

    def vbody(c):
        lo, hi = c
        mid = (lo & hi) + ((lo ^ hi) >> 1)
        cnt = (jnp.sum((key >= mid).astype(jnp.int32), axis=1, keepdims=True)
               + (keyw >= mid).astype(jnp.int32))
        ge = cnt >= _K
        return jnp.where(ge, mid, lo), jnp.where(ge, hi, mid)

    lo, _ = jax.lax.while_loop(vcond, vbody, (lo0, hi0))
    m64 = lo

    gt = key > m64
    eq = key == m64
    gtw = keyw > m64
    eqw = keyw == m64
    cnt_gt = (jnp.sum(gt.astype(jnp.int32), axis=1, keepdims=True)
              + gtw.astype(jnp.int32))
    cnt_eq = jnp.sum(eq.astype(jnp.int32), axis=1, keepdims=True)
    cnt_ge = cnt_gt + cnt_eq + eqw.astype(jnp.int32)

    idx = jax.lax.broadcasted_iota(jnp.int32, (blk, n), 1)
    loi0 = jnp.where(cnt_ge == _K, jnp.int32(n), jnp.int32(0))
    hii0 = jnp.full((blk, 1), n + 1, jnp.int32)

    def icond(c):
        lo_i, hi_i = c
        return jnp.any(hi_i - lo_i > 1)

    def ibody(c):
        lo_i, hi_i = c
        mid = lo_i + (hi_i - lo_i) // 2
        f = cnt_gt + jnp.sum((eq & (idx < mid)).astype(jnp.int32),
                             axis=1, keepdims=True)
        ge = f >= _K
        return jnp.where(ge, lo_i, mid), jnp.where(ge, mid, hi_i)

    _, istar = jax.lax.while_loop(icond, ibody, (loi0, hii0))

    op_ref[...] = gt | (eq & (idx < istar))
    wm_ref[...] = gtw | (eqw & (istar == n + 1))


def kernel(slot_scores, write_score):
    b, n = slot_scores.shape
    w2d = write_score.reshape(b, 1)
    grid = b // _BLK
    op_mask, wm2d = pl.pallas_call(
        _select_kernel,
        grid=(grid,),
        in_specs=[
            pl.BlockSpec((_BLK, n), lambda i: (i, 0)),
            pl.BlockSpec((_BLK, 1), lambda i: (i, 0)),
        ],
        out_specs=[
            pl.BlockSpec((_BLK, n), lambda i: (i, 0)),
            pl.BlockSpec((_BLK, 1), lambda i: (i, 0)),
        ],
        out_shape=[
            jax.ShapeDtypeStruct((b, n), jnp.bool_),
            jax.ShapeDtypeStruct((b, 1), jnp.bool_),
        ],
    )(slot_scores, w2d)
    return op_mask, wm2d.reshape(b)
